# baseline (device time: 848255 ns/iter reference)
import jax
import jax.numpy as jnp
from jax import lax
from jax.experimental import pallas as pl
from jax.experimental.pallas import tpu as pltpu

N_CHUNKS = 32


def kernel(x):
    m, n = x.shape
    half = m // 2
    rpc = half // N_CHUNKS

    def body(x_ref, out_ref, recv_ref, send_y, recv_y, send_x, recv_x,
             va0, vb0, vo0, va1, vb1, vo1, va2, vb2, vo2, local_sems):
        my_x = lax.axis_index("x")
        my_y = lax.axis_index("y")
        my_z = lax.axis_index("z")
        y_nbr = (my_x, 1 - my_y, my_z)
        x_nbr = (1 - my_x, my_y, my_z)

        barrier = pltpu.get_barrier_semaphore()
        for nbr in (y_nbr, x_nbr):
            pl.semaphore_signal(
                barrier, inc=1, device_id=nbr,
                device_id_type=pl.DeviceIdType.MESH,
            )
        pl.semaphore_wait(barrier, 2)

        my_off = my_x * half
        other_off = (1 - my_x) * half

        def my_rows(k):
            return pl.ds(my_off + k * rpc, rpc)

        def other_rows(k):
            return pl.ds(other_off + k * rpc, rpc)

        y_send = [
            pltpu.make_async_remote_copy(
                src_ref=x_ref.at[my_rows(k), :],
                dst_ref=recv_ref.at[my_rows(k), :],
                send_sem=send_y.at[k],
                recv_sem=recv_y.at[k],
                device_id=y_nbr,
                device_id_type=pl.DeviceIdType.MESH,
            )
            for k in range(N_CHUNKS)
        ]
        x_fwd = [
            pltpu.make_async_remote_copy(
                src_ref=recv_ref.at[my_rows(k), :],
                dst_ref=recv_ref.at[my_rows(k), :],
                send_sem=send_x.at[k],
                recv_sem=recv_x.at[k],
                device_id=x_nbr,
                device_id_type=pl.DeviceIdType.MESH,
            )
            for k in range(N_CHUNKS)
        ]
        x_recv = [
            pltpu.make_async_remote_copy(
                src_ref=recv_ref.at[other_rows(k), :],
                dst_ref=recv_ref.at[other_rows(k), :],
                send_sem=send_x.at[k],
                recv_sem=recv_x.at[k],
                device_id=x_nbr,
                device_id_type=pl.DeviceIdType.MESH,
            )
            for k in range(N_CHUNKS)
        ]

        va = [va0, va1, va2]
        vb = [vb0, vb1, vb2]
        vo = [vo0, vo1, vo2]
        NSETS = 3
        prev_store = [None, None, None]
        turn = [0]
        pending = []

        def finish_one():
            s, r, ca, cb = pending.pop(0)
            if prev_store[s] is not None:
                prev_store[s].wait()
            ca.wait()
            cb.wait()
            vo[s][...] = va[s][...] + vb[s][...]
            co = pltpu.make_async_copy(
                vo[s], out_ref.at[r, :], local_sems.at[s, 2])
            co.start()
            prev_store[s] = co

        def process2(base, k0):
            if len(pending) == NSETS:
                finish_one()
            s = turn[0]
            turn[0] = (turn[0] + 1) % NSETS
            r = pl.ds(base + k0 * rpc, 2 * rpc)
            ca = pltpu.make_async_copy(
                x_ref.at[r, :], va[s], local_sems.at[s, 0])
            cb = pltpu.make_async_copy(
                recv_ref.at[r, :], vb[s], local_sems.at[s, 1])
            ca.start()
            cb.start()
            pending.append((s, r, ca, cb))

        for k in range(N_CHUNKS):
            y_send[k].start()
        for k in range(N_CHUNKS):
            y_send[k].wait_recv()
            x_fwd[k].start()
            if k >= 1:
                x_recv[k - 1].wait_recv()
            if k % 2 == 1:
                process2(my_off, k - 1)
                if k >= 3:
                    process2(other_off, k - 3)
        x_recv[N_CHUNKS - 1].wait_recv()
        process2(other_off, N_CHUNKS - 2)

        while pending:
            finish_one()
        for s in range(NSETS):
            if prev_store[s] is not None:
                prev_store[s].wait()
        for k in range(N_CHUNKS):
            y_send[k].wait_send()
            x_fwd[k].wait_send()

    out, _ = pl.pallas_call(
        body,
        out_shape=(
            jax.ShapeDtypeStruct((m, n), x.dtype),
            jax.ShapeDtypeStruct((m, n), x.dtype),
        ),
        in_specs=[pl.BlockSpec(memory_space=pl.ANY)],
        out_specs=(
            pl.BlockSpec(memory_space=pl.ANY),
            pl.BlockSpec(memory_space=pl.ANY),
        ),
        scratch_shapes=[
            pltpu.SemaphoreType.DMA((N_CHUNKS,)),
            pltpu.SemaphoreType.DMA((N_CHUNKS,)),
            pltpu.SemaphoreType.DMA((N_CHUNKS,)),
            pltpu.SemaphoreType.DMA((N_CHUNKS,)),
            pltpu.VMEM((2 * rpc, n), x.dtype),
            pltpu.VMEM((2 * rpc, n), x.dtype),
            pltpu.VMEM((2 * rpc, n), x.dtype),
            pltpu.VMEM((2 * rpc, n), x.dtype),
            pltpu.VMEM((2 * rpc, n), x.dtype),
            pltpu.VMEM((2 * rpc, n), x.dtype),
            pltpu.VMEM((2 * rpc, n), x.dtype),
            pltpu.VMEM((2 * rpc, n), x.dtype),
            pltpu.VMEM((2 * rpc, n), x.dtype),
            pltpu.SemaphoreType.DMA((3, 3)),
        ],
        compiler_params=pltpu.CompilerParams(
            collective_id=0,
            vmem_limit_bytes=100 * 1024 * 1024,
        ),
    )(x)
    return out


# device time: 716216 ns/iter; 1.1844x vs baseline; 1.1844x over previous
import jax
import jax.numpy as jnp
from jax import lax
from jax.experimental import pallas as pl
from jax.experimental.pallas import tpu as pltpu

QC = 16
YD = 4
XG = 6
ZG = 6


def kernel(x):
    m, n = x.shape
    qrows = m // 4
    rpc = qrows // QC

    def body(x_ref, out_ref, recv_ref,
             send_y, recv_y, send_xd, recv_xd, send_zd, recv_zd,
             send_xg, recv_xg, send_zg, recv_zg,
             va0, vb0, vo0, va1, vb1, vo1, va2, vb2, vo2, local_sems):
        X = lax.axis_index("x")
        Y = lax.axis_index("y")
        Z = lax.axis_index("z")
        y_nbr = (X, 1 - Y, Z)
        x_nbr = (1 - X, Y, Z)
        z_nbr = (X, Y, 1 - Z)

        barrier = pltpu.get_barrier_semaphore()
        for nbr in (y_nbr, x_nbr, z_nbr):
            pl.semaphore_signal(
                barrier, inc=1, device_id=nbr,
                device_id_type=pl.DeviceIdType.MESH,
            )
        pl.semaphore_wait(barrier, 3)

        q_me = (X * 2 + Z) * qrows
        q_x = ((1 - X) * 2 + Z) * qrows
        q_z = (X * 2 + (1 - Z)) * qrows
        q_diag = ((1 - X) * 2 + (1 - Z)) * qrows

        def rows(base, k):
            return pl.ds(base + k * rpc, rpc)

        def rdma(src_rows, dst_rows, ssem, rsem, dev):
            return pltpu.make_async_remote_copy(
                src_ref=recv_ref.at[src_rows, :],
                dst_ref=recv_ref.at[dst_rows, :],
                send_sem=ssem, recv_sem=rsem,
                device_id=dev, device_id_type=pl.DeviceIdType.MESH,
            )

        def y_src_rows(k):
            return rows(q_me, k) if k < QC else rows(q_diag, k - QC)

        y_send = [
            pltpu.make_async_remote_copy(
                src_ref=x_ref.at[y_src_rows(k), :],
                dst_ref=recv_ref.at[y_src_rows(k), :],
                send_sem=send_y.at[k], recv_sem=recv_y.at[k],
                device_id=y_nbr, device_id_type=pl.DeviceIdType.MESH,
            )
            for k in range(QC + YD)
        ]
        xd_fwd = [rdma(rows(q_me, k), rows(q_me, k),
                       send_xd.at[k], recv_xd.at[k], x_nbr)
                  for k in range(QC)]
        zd_fwd = [rdma(rows(q_me, k), rows(q_me, k),
                       send_zd.at[k], recv_zd.at[k], z_nbr)
                  for k in range(QC)]
        xd_recv = [rdma(rows(q_x, k), rows(q_x, k),
                        send_xd.at[k], recv_xd.at[k], x_nbr)
                   for k in range(QC)]
        zd_recv = [rdma(rows(q_z, k), rows(q_z, k),
                        send_zd.at[k], recv_zd.at[k], z_nbr)
                   for k in range(QC)]
        xg_fwd = [rdma(rows(q_z, YD + j), rows(q_z, YD + j),
                       send_xg.at[j], recv_xg.at[j], x_nbr)
                  for j in range(XG)]
        zg_fwd = [rdma(rows(q_x, YD + XG + j), rows(q_x, YD + XG + j),
                       send_zg.at[j], recv_zg.at[j], z_nbr)
                  for j in range(ZG)]
        xg_recv = [rdma(rows(q_diag, YD + j), rows(q_diag, YD + j),
                        send_xg.at[j], recv_xg.at[j], x_nbr)
                   for j in range(XG)]
        zg_recv = [rdma(rows(q_diag, YD + XG + j), rows(q_diag, YD + XG + j),
                        send_zg.at[j], recv_zg.at[j], z_nbr)
                   for j in range(ZG)]

        va = [va0, va1, va2]
        vb = [vb0, vb1, vb2]
        vo = [vo0, vo1, vo2]
        NSETS = 3
        prev_store = [None, None, None]
        turn = [0]
        pending = []

        def finish_one():
            s, r, ca, cb = pending.pop(0)
            if prev_store[s] is not None:
                prev_store[s].wait()
            ca.wait()
            cb.wait()
            vo[s][...] = va[s][...] + vb[s][...]
            co = pltpu.make_async_copy(
                vo[s], out_ref.at[r, :], local_sems.at[s, 2])
            co.start()
            prev_store[s] = co

        def process2(base, k0):
            if len(pending) == NSETS:
                finish_one()
            s = turn[0]
            turn[0] = (turn[0] + 1) % NSETS
            r = pl.ds(base + k0 * rpc, 2 * rpc)
            ca = pltpu.make_async_copy(
                x_ref.at[r, :], va[s], local_sems.at[s, 0])
            cb = pltpu.make_async_copy(
                recv_ref.at[r, :], vb[s], local_sems.at[s, 1])
            ca.start()
            cb.start()
            pending.append((s, r, ca, cb))

        for k in range(QC + YD):
            y_send[k].start()

        for k in range(QC):
            y_send[k].wait_recv()
            xd_fwd[k].start()
            zd_fwd[k].start()
            if k >= 1:
                xd_recv[k - 1].wait_recv()
                if k - 1 >= YD + XG:
                    zg_fwd[k - 1 - (YD + XG)].start()
                zd_recv[k - 1].wait_recv()
                if YD <= k - 1 < YD + XG:
                    xg_fwd[k - 1 - YD].start()
            if k % 2 == 1:
                process2(q_me, k - 1)
                if k >= 3:
                    process2(q_x, k - 3)
                    process2(q_z, k - 3)
        xd_recv[QC - 1].wait_recv()
        zg_fwd[ZG - 1].start()
        zd_recv[QC - 1].wait_recv()
        process2(q_x, QC - 2)
        process2(q_z, QC - 2)

        for i in range(YD):
            y_send[QC + i].wait_recv()
        for i in range(0, YD, 2):
            process2(q_diag, i)
        for j in range(XG):
            xg_recv[j].wait_recv()
            if j % 2 == 1:
                process2(q_diag, YD + j - 1)
        for j in range(ZG):
            zg_recv[j].wait_recv()
            if j % 2 == 1:
                process2(q_diag, YD + XG + j - 1)

        while pending:
            finish_one()
        for s in range(NSETS):
            if prev_store[s] is not None:
                prev_store[s].wait()
        for k in range(QC + YD):
            y_send[k].wait_send()
        for k in range(QC):
            xd_fwd[k].wait_send()
            zd_fwd[k].wait_send()
        for j in range(XG):
            xg_fwd[j].wait_send()
        for j in range(ZG):
            zg_fwd[j].wait_send()

    out, _ = pl.pallas_call(
        body,
        out_shape=(
            jax.ShapeDtypeStruct((m, n), x.dtype),
            jax.ShapeDtypeStruct((m, n), x.dtype),
        ),
        in_specs=[pl.BlockSpec(memory_space=pl.ANY)],
        out_specs=(
            pl.BlockSpec(memory_space=pl.ANY),
            pl.BlockSpec(memory_space=pl.ANY),
        ),
        scratch_shapes=[
            pltpu.SemaphoreType.DMA((QC + YD,)),
            pltpu.SemaphoreType.DMA((QC + YD,)),
            pltpu.SemaphoreType.DMA((QC,)),
            pltpu.SemaphoreType.DMA((QC,)),
            pltpu.SemaphoreType.DMA((QC,)),
            pltpu.SemaphoreType.DMA((QC,)),
            pltpu.SemaphoreType.DMA((XG,)),
            pltpu.SemaphoreType.DMA((XG,)),
            pltpu.SemaphoreType.DMA((ZG,)),
            pltpu.SemaphoreType.DMA((ZG,)),
            pltpu.VMEM((2 * rpc, n), x.dtype),
            pltpu.VMEM((2 * rpc, n), x.dtype),
            pltpu.VMEM((2 * rpc, n), x.dtype),
            pltpu.VMEM((2 * rpc, n), x.dtype),
            pltpu.VMEM((2 * rpc, n), x.dtype),
            pltpu.VMEM((2 * rpc, n), x.dtype),
            pltpu.VMEM((2 * rpc, n), x.dtype),
            pltpu.VMEM((2 * rpc, n), x.dtype),
            pltpu.VMEM((2 * rpc, n), x.dtype),
            pltpu.SemaphoreType.DMA((3, 3)),
        ],
        compiler_params=pltpu.CompilerParams(
            collective_id=0,
            vmem_limit_bytes=100 * 1024 * 1024,
        ),
    )(x)
    return out
